# baseline (speedup 1.0000x reference)
import functools

import jax
import jax.numpy as jnp
from jax import lax
from jax.experimental import pallas as pl
from jax.experimental.pallas import tpu as pltpu
from jax.experimental.pallas import tpu_sc as plsc

D = 128
HALF = 64
HW = 900
N_PIX = 256 * HW
NUM_COLORS = 10
NW = 32
PPW = N_PIX // NW
CHUNK = 240
N_CHUNKS = PPW // CHUNK
GROUPS = CHUNK // 16


def _sc_kernel(x_hbm, idx_hbm, sp_hbm, ch_hbm, out_hbm,
               x_v, out_v, idx_v, sp_v, ch_v):
    wid = lax.axis_index("s") * 2 + lax.axis_index("c")
    base = wid * PPW

    pltpu.sync_copy(sp_hbm, sp_v)
    pltpu.sync_copy(ch_hbm, ch_v)

    lane = lax.iota(jnp.int32, 16)

    def chunk_body(c, _):
        pstart = base + c * CHUNK
        pltpu.sync_copy(x_hbm.at[pl.ds(pstart * D, CHUNK * D)], x_v)
        pltpu.sync_copy(idx_hbm.at[pl.ds(pstart, CHUNK)], idx_v)

        def group_body(g, _):
            p16 = g * 16
            pglob = pstart + p16 + lane
            spbase = lax.rem(pglob, HW) * HALF
            cidx = idx_v[pl.ds(p16, 16)]
            chbase = cidx * HALF
            xbase = (p16 + lane) * D
            for jj in range(HALF):
                xi0 = xbase + jj
                v0 = (plsc.load_gather(x_v, [xi0])
                      + plsc.load_gather(sp_v, [spbase + jj]))
                plsc.store_scatter(out_v, [xi0], v0)
                xi1 = xbase + (HALF + jj)
                v1 = (plsc.load_gather(x_v, [xi1])
                      + plsc.load_gather(ch_v, [chbase + jj]))
                plsc.store_scatter(out_v, [xi1], v1)
            return 0

        lax.fori_loop(0, GROUPS, group_body, 0)
        pltpu.sync_copy(out_v, out_hbm.at[pl.ds(pstart * D, CHUNK * D)])
        return 0

    lax.fori_loop(0, N_CHUNKS, chunk_body, 0)


def kernel(x, color_indices, spatial_pe, chromatic_pe):
    Bb, Hh, Ww, d = x.shape
    xf = x.reshape(N_PIX * D)
    idxf = color_indices.astype(jnp.int32).reshape(N_PIX)
    spf = spatial_pe[:Hh, :Ww, :].reshape(HW * HALF)
    chf = chromatic_pe.reshape(NUM_COLORS * HALF)

    mesh = plsc.VectorSubcoreMesh(core_axis_name="c", subcore_axis_name="s")
    run = pl.kernel(
        _sc_kernel,
        jax.ShapeDtypeStruct((N_PIX * D,), jnp.float32),
        mesh=mesh,
        compiler_params=pltpu.CompilerParams(needs_layout_passes=False),
        scratch_types=[
            pltpu.VMEM((CHUNK * D,), jnp.float32),
            pltpu.VMEM((CHUNK * D,), jnp.float32),
            pltpu.VMEM((CHUNK,), jnp.int32),
            pltpu.VMEM((HW * HALF,), jnp.float32),
            pltpu.VMEM((NUM_COLORS * HALF,), jnp.float32),
        ],
    )
    out = run(xf, idxf, spf, chf)
    return out.reshape(Bb, Hh, Ww, d)


# speedup vs baseline: 2.6386x; 2.6386x over previous
import functools

import jax
import jax.numpy as jnp
from jax import lax
from jax.experimental import pallas as pl
from jax.experimental.pallas import tpu as pltpu
from jax.experimental.pallas import tpu_sc as plsc

D = 128
HALF = 64
HW = 900
N_PIX = 256 * HW
NUM_COLORS = 10
NW = 32
PPW = N_PIX // NW
CHUNK = 240
N_CHUNKS = PPW // CHUNK
GROUPS = CHUNK // 16


def _sc_kernel(x_hbm, idx_hbm, sp_hbm, ch_hbm, out_hbm,
               x_v, out_v, idx_v, sp_v, ch_v):
    wid = lax.axis_index("s") * 2 + lax.axis_index("c")
    base = wid * PPW

    pltpu.sync_copy(sp_hbm, sp_v)
    pltpu.sync_copy(ch_hbm, ch_v)

    lane = lax.iota(jnp.int32, 16)

    def chunk_body(c, _):
        pstart = base + c * CHUNK
        pltpu.sync_copy(x_hbm.at[pl.ds(pstart * D, CHUNK * D)], x_v)
        pltpu.sync_copy(idx_hbm.at[pl.ds(pstart, CHUNK)], idx_v)

        def group_body(g, _):
            p16 = g * 16
            cidx = idx_v[pl.ds(p16, 16)]
            for p in range(16):
                ploc = p16 + p
                spoff = lax.rem(pstart + ploc, HW) * HALF
                xoff = ploc * D
                for j in range(HALF // 16):
                    o = j * 16
                    out_v[pl.ds(xoff + o, 16)] = (
                        x_v[pl.ds(xoff + o, 16)]
                        + sp_v[pl.ds(spoff + o, 16)])
                cbase = cidx[jnp.full((16,), p, jnp.int32)] * HALF
                for j in range(HALF // 16):
                    o = j * 16
                    cv = plsc.load_gather(ch_v, [cbase + o + lane])
                    out_v[pl.ds(xoff + HALF + o, 16)] = (
                        x_v[pl.ds(xoff + HALF + o, 16)] + cv)
            return 0

        lax.fori_loop(0, GROUPS, group_body, 0)
        pltpu.sync_copy(out_v, out_hbm.at[pl.ds(pstart * D, CHUNK * D)])
        return 0

    lax.fori_loop(0, N_CHUNKS, chunk_body, 0)


def kernel(x, color_indices, spatial_pe, chromatic_pe):
    Bb, Hh, Ww, d = x.shape
    xf = x.reshape(N_PIX * D)
    idxf = color_indices.astype(jnp.int32).reshape(N_PIX)
    spf = spatial_pe[:Hh, :Ww, :].reshape(HW * HALF)
    chf = chromatic_pe.reshape(NUM_COLORS * HALF)

    mesh = plsc.VectorSubcoreMesh(core_axis_name="c", subcore_axis_name="s")
    run = pl.kernel(
        _sc_kernel,
        jax.ShapeDtypeStruct((N_PIX * D,), jnp.float32),
        mesh=mesh,
        compiler_params=pltpu.CompilerParams(needs_layout_passes=False),
        scratch_types=[
            pltpu.VMEM((CHUNK * D,), jnp.float32),
            pltpu.VMEM((CHUNK * D,), jnp.float32),
            pltpu.VMEM((CHUNK,), jnp.int32),
            pltpu.VMEM((HW * HALF,), jnp.float32),
            pltpu.VMEM((NUM_COLORS * HALF,), jnp.float32),
        ],
    )
    out = run(xf, idxf, spf, chf)
    return out.reshape(Bb, Hh, Ww, d)


# speedup vs baseline: 2.8808x; 1.0918x over previous
import jax
import jax.numpy as jnp
from jax import lax
from jax.experimental import pallas as pl
from jax.experimental.pallas import tpu as pltpu
from jax.experimental.pallas import tpu_sc as plsc

D = 128
HALF = 64
HW = 900
N_PIX = 256 * HW
NUM_COLORS = 10
NW = 32
PPW = N_PIX // NW
CHUNK = 160
N_CHUNKS = PPW // CHUNK
NBUF = 3
GROUPS = CHUNK // 16


def _sc_kernel(x_hbm, idx_hbm, sp_hbm, ch_hbm, out_hbm,
               x_v0, x_v1, x_v2, i_v0, i_v1, i_v2, sp_v, ch_v,
               ld0, ld1, ld2, st0, st1, st2):
    x_bufs = (x_v0, x_v1, x_v2)
    i_bufs = (i_v0, i_v1, i_v2)
    ld_sems = (ld0, ld1, ld2)
    st_sems = (st0, st1, st2)

    wid = lax.axis_index("s") * 2 + lax.axis_index("c")
    base = wid * PPW

    pltpu.sync_copy(sp_hbm, sp_v)
    pltpu.sync_copy(ch_hbm, ch_v)

    lane = lax.iota(jnp.int32, 16)

    def start_load(c, b):
        pstart = base + c * CHUNK
        pltpu.async_copy(x_hbm.at[pl.ds(pstart * D, CHUNK * D)],
                         x_bufs[b], ld_sems[b])
        pltpu.async_copy(idx_hbm.at[pl.ds(pstart, CHUNK)],
                         i_bufs[b], ld_sems[b])

    def wait_load(c, b):
        pstart = base + c * CHUNK
        pltpu.make_async_copy(x_hbm.at[pl.ds(pstart * D, CHUNK * D)],
                              x_bufs[b], ld_sems[b]).wait()
        pltpu.make_async_copy(idx_hbm.at[pl.ds(pstart, CHUNK)],
                              i_bufs[b], ld_sems[b]).wait()

    def start_store(c, b):
        pstart = base + c * CHUNK
        pltpu.async_copy(x_bufs[b],
                         out_hbm.at[pl.ds(pstart * D, CHUNK * D)],
                         st_sems[b])

    def wait_store(c, b):
        pstart = base + c * CHUNK
        pltpu.make_async_copy(x_bufs[b],
                              out_hbm.at[pl.ds(pstart * D, CHUNK * D)],
                              st_sems[b]).wait()

    for b in range(NBUF):
        start_load(b, b)

    def chunk_compute(pstart, x_v, idx_v):
        def group_body(g, _):
            p16 = g * 16
            cidx = idx_v[pl.ds(p16, 16)]
            for p in range(16):
                ploc = p16 + p
                spoff = lax.rem(pstart + ploc, HW) * HALF
                xoff = ploc * D
                for j in range(HALF // 16):
                    o = j * 16
                    x_v[pl.ds(xoff + o, 16)] = (
                        x_v[pl.ds(xoff + o, 16)]
                        + sp_v[pl.ds(spoff + o, 16)])
                cbase = cidx[jnp.full((16,), p, jnp.int32)] * HALF
                for j in range(HALF // 16):
                    o = j * 16
                    cv = plsc.load_gather(ch_v, [cbase + o + lane])
                    x_v[pl.ds(xoff + HALF + o, 16)] = (
                        x_v[pl.ds(xoff + HALF + o, 16)] + cv)
            return 0

        lax.fori_loop(0, GROUPS, group_body, 0)

    def iter_body(k, _):
        for b in range(NBUF):
            c = k * NBUF + b
            wait_load(c, b)
            chunk_compute(base + c * CHUNK, x_bufs[b], i_bufs[b])
            start_store(c, b)
            @pl.when(c + NBUF < N_CHUNKS)
            def _():
                wait_store(c, b)
                start_load(c + NBUF, b)
        return 0

    lax.fori_loop(0, N_CHUNKS // NBUF, iter_body, 0)

    for b in range(NBUF):
        wait_store(N_CHUNKS - NBUF + b, b)


def kernel(x, color_indices, spatial_pe, chromatic_pe):
    Bb, Hh, Ww, d = x.shape
    xf = x.reshape(N_PIX * D)
    idxf = color_indices.astype(jnp.int32).reshape(N_PIX)
    spf = spatial_pe[:Hh, :Ww, :].reshape(HW * HALF)
    chf = chromatic_pe.reshape(NUM_COLORS * HALF)

    mesh = plsc.VectorSubcoreMesh(core_axis_name="c", subcore_axis_name="s")
    run = pl.kernel(
        _sc_kernel,
        jax.ShapeDtypeStruct((N_PIX * D,), jnp.float32),
        mesh=mesh,
        compiler_params=pltpu.CompilerParams(needs_layout_passes=False),
        scratch_types=[
            pltpu.VMEM((CHUNK * D,), jnp.float32),
            pltpu.VMEM((CHUNK * D,), jnp.float32),
            pltpu.VMEM((CHUNK * D,), jnp.float32),
            pltpu.VMEM((CHUNK,), jnp.int32),
            pltpu.VMEM((CHUNK,), jnp.int32),
            pltpu.VMEM((CHUNK,), jnp.int32),
            pltpu.VMEM((HW * HALF,), jnp.float32),
            pltpu.VMEM((NUM_COLORS * HALF,), jnp.float32),
            pltpu.SemaphoreType.DMA,
            pltpu.SemaphoreType.DMA,
            pltpu.SemaphoreType.DMA,
            pltpu.SemaphoreType.DMA,
            pltpu.SemaphoreType.DMA,
            pltpu.SemaphoreType.DMA,
        ],
    )
    out = run(xf, idxf, spf, chf)
    return out.reshape(Bb, Hh, Ww, d)
